# baseline (speedup 1.0000x reference)
import functools

import jax
import jax.numpy as jnp
from jax import lax
from jax.experimental import pallas as pl
from jax.experimental.pallas import tpu as pltpu
from jax.experimental.pallas import tpu_sc as plsc

NS = 16
CK = 128
GS = 16


def _leaky(v):
    return jnp.where(v >= 0, v, 0.01 * v)


def _make_deg_kernel(n_pad, e_rows):
    rows_t = n_pad // NS
    ert = e_rows // NS
    mesh = plsc.VectorSubcoreMesh(core_axis_name="c", subcore_axis_name="s")
    f32 = jnp.float32

    def body(dst1, dst2, d1, d2, idx0_v, idx1_v, tab_v, red_v, degb_v, red_sh,
             sem0, sem1):
        s = lax.axis_index("s")
        c = lax.axis_index("c")

        def run(dst, d_out):
            def z(i, carry):
                tab_v[pl.ds(i * 16, 16)] = jnp.zeros((16,), f32)
                return carry

            lax.fori_loop(0, n_pad // 16, z, 0)
            base = s * ert
            pltpu.sync_copy(dst.at[base], idx0_v)
            pltpu.async_copy(dst.at[base + 1], idx1_v, sem1)

            def scat(idx_v):
                def sc16(i, carry2):
                    iv = idx_v[pl.ds(i * 16, 16)]
                    plsc.addupdate_scatter(tab_v, [iv],
                                           jnp.ones((16,), f32))
                    return carry2

                lax.fori_loop(0, CK // 16, sc16, 0)

            def pair(t, carry):
                c0 = 2 * t
                scat(idx0_v)

                @pl.when(c0 + 2 < ert)
                def _():
                    pltpu.async_copy(dst.at[base + c0 + 2], idx0_v, sem0)

                pltpu.make_async_copy(dst.at[base], idx1_v, sem1).wait()
                scat(idx1_v)

                @pl.when(c0 + 3 < ert)
                def _():
                    pltpu.async_copy(dst.at[base + c0 + 3], idx1_v, sem1)

                @pl.when(c0 + 2 < ert)
                def _():
                    pltpu.make_async_copy(dst.at[base], idx0_v, sem0).wait()

                return carry

            lax.fori_loop(0, ert // 2, pair, 0)
            pltpu.sync_copy(tab_v, red_sh.at[s])
            plsc.subcore_barrier()
            pltpu.sync_copy(red_sh.at[:, pl.ds(s * rows_t, rows_t)], red_v)

            def red(cix, carry):
                acc = jnp.zeros((16,), f32)
                for t in range(NS):
                    acc = acc + red_v[t, pl.ds(cix * 16, 16)]
                tab_v[pl.ds(cix * 16, 16)] = acc
                return carry

            lax.fori_loop(0, rows_t // 16, red, 0)

            def bc(nix, carry):
                vec = plsc.load_gather(tab_v, [jnp.full((16,), nix,
                                                        jnp.int32)])
                for cix in range(8):
                    degb_v[nix, pl.ds(cix * 16, 16)] = vec
                return carry

            lax.fori_loop(0, rows_t, bc, 0)
            pltpu.sync_copy(degb_v, d_out.at[pl.ds(s * rows_t, rows_t)])

        @pl.when(c == 0)
        def _():
            run(dst1, d1)

        @pl.when(c == 1)
        def _():
            run(dst2, d2)

    return pl.kernel(
        body,
        out_type=[jax.ShapeDtypeStruct((n_pad, 128), jnp.float32)] * 2,
        mesh=mesh,
        scratch_types=[
            pltpu.VMEM((CK,), jnp.int32),
            pltpu.VMEM((CK,), jnp.int32),
            pltpu.VMEM((n_pad,), jnp.float32),
            pltpu.VMEM((NS, n_pad // NS), jnp.float32),
            pltpu.VMEM((n_pad // NS, 128), jnp.float32),
            pltpu.VMEM_SHARED((NS, n_pad), jnp.float32),
            pltpu.SemaphoreType.DMA,
            pltpu.SemaphoreType.DMA,
        ],
        compiler_params=pltpu.CompilerParams(needs_layout_passes=False),
    )


def _make_agg_kernel(n_pad, e_rows, d):
    rows_t = n_pad // NS
    ert = e_rows // NS
    n_out = ert // GS
    mesh = plsc.VectorSubcoreMesh(core_axis_name="c", subcore_axis_name="s")

    NB = 2
    NI = 4

    def body(src1, dst1, y1, src2, dst2, y2, agg1, agg2, sidx, didx,
             rows, agg_sh, sem_g, sem_s, sem_i):
        s = lax.axis_index("s")
        c = lax.axis_index("c")

        def run(src, dst, y, agg_out):
            pltpu.sync_copy(y.at[pl.ds(s * rows_t, rows_t)],
                            agg_sh.at[pl.ds(s * rows_t, rows_t)])
            plsc.subcore_barrier()

            base = s * ert
            for k in range(NB):
                pltpu.sync_copy(src.at[base + k], sidx[k])
                pltpu.sync_copy(dst.at[base + k], didx[k])
                pltpu.async_copy(y.at[sidx[k]], rows[k], sem_g[k])

            def octet(t, carry):
                for k in range(NI):
                    ck = NI * t + k
                    srot = k % NB
                    prot = (k + NB) % NI
                    pltpu.make_async_copy(y.at[sidx[k]], rows[srot],
                                          sem_g[srot]).wait()
                    sc = pltpu.async_copy(rows[srot], agg_sh.at[didx[k]],
                                          sem_s, add=True)

                    @pl.when(ck + NB < ert)
                    def _():
                        pltpu.async_copy(src.at[base + ck + NB], sidx[prot],
                                         sem_i)
                        pltpu.async_copy(dst.at[base + ck + NB], didx[prot],
                                         sem_i)

                    sc.wait()

                    @pl.when(ck + NB < ert)
                    def _():
                        pltpu.make_async_copy(src.at[base], sidx[prot],
                                              sem_i).wait()
                        pltpu.make_async_copy(dst.at[base], didx[prot],
                                              sem_i).wait()
                        pltpu.async_copy(y.at[sidx[prot]], rows[srot],
                                         sem_g[srot])

                return carry

            lax.fori_loop(0, ert // NI, octet, 0)
            plsc.subcore_barrier()
            pltpu.sync_copy(agg_sh.at[pl.ds(s * rows_t, rows_t)],
                            agg_out.at[pl.ds(s * rows_t, rows_t)])

        @pl.when(c == 0)
        def _():
            run(src1, dst1, y1, agg1)

        @pl.when(c == 1)
        def _():
            run(src2, dst2, y2, agg2)

    return pl.kernel(
        body,
        out_type=[jax.ShapeDtypeStruct((n_pad, d), jnp.float32)] * 2,
        mesh=mesh,
        scratch_types=[
            [pltpu.VMEM((CK,), jnp.int32)] * NI,
            [pltpu.VMEM((CK,), jnp.int32)] * NI,
            [pltpu.VMEM((CK, d), jnp.float32)] * NB,
            pltpu.VMEM_SHARED((n_pad, d), jnp.float32),
            [pltpu.SemaphoreType.DMA] * NB,
            pltpu.SemaphoreType.DMA,
            pltpu.SemaphoreType.DMA,
        ],
    )


def _xw_body(x1_ref, w1_ref, x2_ref, w2_ref, y1_ref, y2_ref):
    y1_ref[...] = jnp.dot(x1_ref[...], w1_ref[...],
                          preferred_element_type=jnp.float32)
    y2_ref[...] = jnp.dot(x2_ref[...], w2_ref[...],
                          preferred_element_type=jnp.float32)


def _make_xw_kernel(n_pad, d, out):
    rb = 512
    grid = (n_pad // rb,)
    row = lambda i: (i, 0)
    fixed = lambda i: (0, 0)
    return pl.pallas_call(
        _xw_body,
        grid=grid,
        in_specs=[
            pl.BlockSpec((rb, d), row),
            pl.BlockSpec((d, out), fixed),
            pl.BlockSpec((rb, d), row),
            pl.BlockSpec((d, out), fixed),
        ],
        out_specs=[
            pl.BlockSpec((rb, out), row),
            pl.BlockSpec((rb, out), row),
        ],
        out_shape=[jax.ShapeDtypeStruct((n_pad, out), jnp.float32)] * 2,
    )


def _scale_body(xw1_ref, h1_ref, xw2_ref, h2_ref, y1_ref, y2_ref):
    y1_ref[...] = xw1_ref[...] * lax.rsqrt(h1_ref[...] + 1.0)
    y2_ref[...] = xw2_ref[...] * lax.rsqrt(h2_ref[...] + 1.0)


def _make_scale_kernel(n_pad, out):
    rb = 512
    row = lambda i: (i, 0)
    return pl.pallas_call(
        _scale_body,
        grid=(n_pad // rb,),
        in_specs=[pl.BlockSpec((rb, out), row)] * 4,
        out_specs=[pl.BlockSpec((rb, out), row)] * 2,
        out_shape=[jax.ShapeDtypeStruct((n_pad, out), jnp.float32)] * 2,
    )


def _make_head_kernel(n_pad, out, n_graphs):
    rb = 512
    nblk = n_pad // rb

    def body(agg1_ref, h1_ref, bv1_ref, agg2_ref, h2_ref, bv2_ref,
             bc1_ref, bc2_ref, wp1_ref, bp1_ref, wp2_ref, bp2_ref,
             w1_ref, b1_ref, w2_ref, b2_ref, wo_ref, bo_ref, o_ref,
             pool1, pool2):
        i = pl.program_id(0)

        @pl.when(i == 0)
        def _():
            pool1[...] = jnp.full((n_graphs + 1, out), -jnp.inf, jnp.float32)
            pool2[...] = jnp.full((n_graphs + 1, out), -jnp.inf, jnp.float32)

        def accum(agg_ref, h_ref, bv_ref, bc_ref, pool):
            dinv = lax.rsqrt(h_ref[...] + 1.0)
            t = _leaky(dinv * agg_ref[...] + bc_ref[...])
            bv = bv_ref[...]
            g_lo = jnp.min(bv)
            g_hi = jnp.max(bv)

            def upd(g, carry):
                m = bv == g
                cand = jnp.max(jnp.where(m, t, -jnp.inf), axis=0,
                               keepdims=True)
                pool[pl.ds(g, 1), :] = jnp.maximum(pool[pl.ds(g, 1), :], cand)
                return carry

            lax.fori_loop(g_lo, g_hi + 1, upd, 0)

        accum(agg1_ref, h1_ref, bv1_ref, bc1_ref, pool1)
        accum(agg2_ref, h2_ref, bv2_ref, bc2_ref, pool2)

        @pl.when(i == nblk - 1)
        def _():
            p1 = pool1[0:n_graphs, :]
            p1 = jnp.where(jnp.isfinite(p1), p1, 0.0)
            p2 = pool2[0:n_graphs, :]
            p2 = jnp.where(jnp.isfinite(p2), p2, 0.0)
            p1 = _leaky(jnp.dot(p1, wp1_ref[...],
                                preferred_element_type=jnp.float32)
                        + bp1_ref[...])
            p2 = _leaky(jnp.dot(p2, wp2_ref[...],
                                preferred_element_type=jnp.float32)
                        + bp2_ref[...])
            cc = jnp.concatenate([p1, p2], axis=1)
            cc = _leaky(jnp.dot(cc, w1_ref[...],
                                preferred_element_type=jnp.float32)
                        + b1_ref[...])
            cc = _leaky(jnp.dot(cc, w2_ref[...],
                                preferred_element_type=jnp.float32)
                        + b2_ref[...])
            z = jnp.dot(cc, wo_ref[...],
                        preferred_element_type=jnp.float32) + bo_ref[...]
            o_ref[...] = 1.0 / (1.0 + jnp.exp(-z))

    row = lambda i: (i, 0)
    fixed = lambda i: (0, 0)
    per_graph = [
        pl.BlockSpec((rb, out), row),
        pl.BlockSpec((rb, 128), row),
        pl.BlockSpec((rb, 1), row),
    ]
    return pl.pallas_call(
        body,
        grid=(nblk,),
        in_specs=per_graph + per_graph + [
            pl.BlockSpec((1, out), fixed),
            pl.BlockSpec((1, out), fixed),
            pl.BlockSpec((out, out), fixed),
            pl.BlockSpec((1, out), fixed),
            pl.BlockSpec((out, out), fixed),
            pl.BlockSpec((1, out), fixed),
            pl.BlockSpec((2 * out, 256), fixed),
            pl.BlockSpec((1, 256), fixed),
            pl.BlockSpec((256, 64), fixed),
            pl.BlockSpec((1, 64), fixed),
            pl.BlockSpec((64, 1), fixed),
            pl.BlockSpec((1, 1), fixed),
        ],
        out_specs=pl.BlockSpec((n_graphs, 1), fixed),
        out_shape=jax.ShapeDtypeStruct((n_graphs, 1), jnp.float32),
        scratch_shapes=[
            pltpu.VMEM((n_graphs + 1, out), jnp.float32),
            pltpu.VMEM((n_graphs + 1, out), jnp.float32),
        ],
    )


def kernel(pro1_x, pro1_edge_index, pro1_batch, pro2_x, pro2_edge_index,
           pro2_batch, W_conv1, b_conv1, W_fc_p1, b_fc_p1, W_conv2, b_conv2,
           W_fc_p2, b_fc_p2, W_fc1, b_fc1, W_fc2, b_fc2, W_out, b_out):
    n, d = pro1_x.shape
    out = W_conv1.shape[1]
    e = pro1_edge_index.shape[1]
    n_graphs = 64

    n_pad = -(-n // (NS * 128)) * (NS * 128)
    scrap = n
    e_pad = -(-e // (NS * CK * GS)) * (NS * CK * GS)
    e_rows = e_pad // CK

    i32 = jnp.int32
    f32 = jnp.float32

    def prep_graph(x, ei, batch):
        ei = ei.astype(i32)
        pad_e = e_pad - e
        src = jnp.concatenate([ei[0], jnp.zeros((pad_e,), i32)])
        dst = jnp.concatenate([ei[1], jnp.full((pad_e,), scrap, i32)])
        src = src.reshape(e_rows, CK)
        dst = dst.reshape(e_rows, CK)
        xp = jnp.concatenate([x, jnp.zeros((n_pad - n, d), f32)], axis=0)
        bp = jnp.concatenate(
            [batch.astype(i32), jnp.full((n_pad - n,), n_graphs, i32)]
        ).reshape(n_pad, 1)
        return src, dst, xp, bp

    src1, dst1, x1p, b1p = prep_graph(pro1_x, pro1_edge_index, pro1_batch)
    src2, dst2, x2p, b2p = prep_graph(pro2_x, pro2_edge_index, pro2_batch)

    xw1, xw2 = _make_xw_kernel(n_pad, d, out)(x1p, W_conv1, x2p, W_conv2)
    h1, h2 = _make_deg_kernel(n_pad, e_rows)(dst1, dst2)
    y1, y2 = _make_scale_kernel(n_pad, out)(xw1, h1, xw2, h2)
    agg1, agg2 = _make_agg_kernel(n_pad, e_rows, out)(src1, dst1, y1, src2,
                                                      dst2, y2)

    res = _make_head_kernel(n_pad, out, n_graphs)(
        agg1, h1, b1p, agg2, h2, b2p,
        b_conv1.reshape(1, out), b_conv2.reshape(1, out),
        W_fc_p1, b_fc_p1.reshape(1, out), W_fc_p2, b_fc_p2.reshape(1, out),
        W_fc1, b_fc1.reshape(1, 256), W_fc2, b_fc2.reshape(1, 64),
        W_out, b_out.reshape(1, 1))
    return res


# speedup vs baseline: 1.1034x; 1.1034x over previous
import functools

import jax
import jax.numpy as jnp
from jax import lax
from jax.experimental import pallas as pl
from jax.experimental.pallas import tpu as pltpu
from jax.experimental.pallas import tpu_sc as plsc

NS = 16
CK = 128
GS = 16


def _leaky(v):
    return jnp.where(v >= 0, v, 0.01 * v)


def _make_deg_kernel(n_pad, e_rows):
    rows_t = n_pad // NS
    ert = e_rows // NS
    mesh = plsc.VectorSubcoreMesh(core_axis_name="c", subcore_axis_name="s")
    f32 = jnp.float32

    def body(dst1, dst2, d1, d2, idx0_v, idx1_v, tab_v, red_v, degb_v, red_sh,
             sem0, sem1):
        s = lax.axis_index("s")
        c = lax.axis_index("c")

        def run(dst, d_out):
            def z(i, carry):
                tab_v[pl.ds(i * 16, 16)] = jnp.zeros((16,), f32)
                return carry

            lax.fori_loop(0, n_pad // 16, z, 0)
            base = s * ert
            pltpu.sync_copy(dst.at[base], idx0_v)
            pltpu.async_copy(dst.at[base + 1], idx1_v, sem1)

            def scat(idx_v):
                def sc16(i, carry2):
                    iv = idx_v[pl.ds(i * 16, 16)]
                    plsc.addupdate_scatter(tab_v, [iv],
                                           jnp.ones((16,), f32))
                    return carry2

                lax.fori_loop(0, CK // 16, sc16, 0)

            def pair(t, carry):
                c0 = 2 * t
                scat(idx0_v)

                @pl.when(c0 + 2 < ert)
                def _():
                    pltpu.async_copy(dst.at[base + c0 + 2], idx0_v, sem0)

                pltpu.make_async_copy(dst.at[base], idx1_v, sem1).wait()
                scat(idx1_v)

                @pl.when(c0 + 3 < ert)
                def _():
                    pltpu.async_copy(dst.at[base + c0 + 3], idx1_v, sem1)

                @pl.when(c0 + 2 < ert)
                def _():
                    pltpu.make_async_copy(dst.at[base], idx0_v, sem0).wait()

                return carry

            lax.fori_loop(0, ert // 2, pair, 0)
            pltpu.sync_copy(tab_v, red_sh.at[s])
            plsc.subcore_barrier()
            pltpu.sync_copy(red_sh.at[:, pl.ds(s * rows_t, rows_t)], red_v)

            def red(cix, carry):
                acc = jnp.zeros((16,), f32)
                for t in range(NS):
                    acc = acc + red_v[t, pl.ds(cix * 16, 16)]
                tab_v[pl.ds(cix * 16, 16)] = acc
                return carry

            lax.fori_loop(0, rows_t // 16, red, 0)

            def bc(nix, carry):
                vec = plsc.load_gather(tab_v, [jnp.full((16,), nix,
                                                        jnp.int32)])
                for cix in range(8):
                    degb_v[nix, pl.ds(cix * 16, 16)] = vec
                return carry

            lax.fori_loop(0, rows_t, bc, 0)
            pltpu.sync_copy(degb_v, d_out.at[pl.ds(s * rows_t, rows_t)])

        @pl.when(c == 0)
        def _():
            run(dst1, d1)

        @pl.when(c == 1)
        def _():
            run(dst2, d2)

    return pl.kernel(
        body,
        out_type=[jax.ShapeDtypeStruct((n_pad, 128), jnp.float32)] * 2,
        mesh=mesh,
        scratch_types=[
            pltpu.VMEM((CK,), jnp.int32),
            pltpu.VMEM((CK,), jnp.int32),
            pltpu.VMEM((n_pad,), jnp.float32),
            pltpu.VMEM((NS, n_pad // NS), jnp.float32),
            pltpu.VMEM((n_pad // NS, 128), jnp.float32),
            pltpu.VMEM_SHARED((NS, n_pad), jnp.float32),
            pltpu.SemaphoreType.DMA,
            pltpu.SemaphoreType.DMA,
        ],
        compiler_params=pltpu.CompilerParams(needs_layout_passes=False),
    )


def _make_agg_kernel(n_pad, e_rows, d):
    rows_t = n_pad // NS
    ert = e_rows // NS
    n_out = ert // GS
    mesh = plsc.VectorSubcoreMesh(core_axis_name="c", subcore_axis_name="s")

    NB = 2
    NI = 4

    def body(src1, dst1, y1, src2, dst2, y2, agg1, agg2, sidx, didx,
             rows, agg_sh, sem_g, sem_s, sem_i):
        s = lax.axis_index("s")
        c = lax.axis_index("c")

        def run(src, dst, y, agg_out):
            pltpu.sync_copy(y.at[pl.ds(s * rows_t, rows_t)],
                            agg_sh.at[pl.ds(s * rows_t, rows_t)])
            plsc.subcore_barrier()

            base = s * ert
            for k in range(NB):
                pltpu.sync_copy(src.at[base + k], sidx[k])
                pltpu.sync_copy(dst.at[base + k], didx[k])
                pltpu.async_copy(y.at[sidx[k]], rows[k], sem_g[k])

            def octet(t, carry):
                for k in range(NI):
                    ck = NI * t + k
                    srot = k % NB
                    prot = (k + NB) % NI
                    pltpu.make_async_copy(y.at[sidx[k]], rows[srot],
                                          sem_g[srot]).wait()
                    sc = pltpu.async_copy(rows[srot], agg_sh.at[didx[k]],
                                          sem_s, add=True)

                    @pl.when(ck + NB < ert)
                    def _():
                        pltpu.async_copy(src.at[base + ck + NB], sidx[prot],
                                         sem_i)
                        pltpu.async_copy(dst.at[base + ck + NB], didx[prot],
                                         sem_i)

                    sc.wait()

                    @pl.when(ck + NB < ert)
                    def _():
                        pltpu.make_async_copy(src.at[base], sidx[prot],
                                              sem_i).wait()
                        pltpu.make_async_copy(dst.at[base], didx[prot],
                                              sem_i).wait()
                        pltpu.async_copy(y.at[sidx[prot]], rows[srot],
                                         sem_g[srot])

                return carry

            lax.fori_loop(0, ert // NI, octet, 0)
            plsc.subcore_barrier()
            pltpu.sync_copy(agg_sh.at[pl.ds(s * rows_t, rows_t)],
                            agg_out.at[pl.ds(s * rows_t, rows_t)])

        @pl.when(c == 0)
        def _():
            run(src1, dst1, y1, agg1)

        @pl.when(c == 1)
        def _():
            run(src2, dst2, y2, agg2)

    return pl.kernel(
        body,
        out_type=[jax.ShapeDtypeStruct((n_pad, d), jnp.float32)] * 2,
        mesh=mesh,
        scratch_types=[
            [pltpu.VMEM((CK,), jnp.int32)] * NI,
            [pltpu.VMEM((CK,), jnp.int32)] * NI,
            [pltpu.VMEM((CK, d), jnp.float32)] * NB,
            pltpu.VMEM_SHARED((n_pad, d), jnp.float32),
            [pltpu.SemaphoreType.DMA] * NB,
            pltpu.SemaphoreType.DMA,
            pltpu.SemaphoreType.DMA,
        ],
    )


def _xw_body(x1_ref, w1_ref, h1_ref, x2_ref, w2_ref, h2_ref, y1_ref, y2_ref):
    d1 = lax.rsqrt(h1_ref[...] + 1.0)
    d2 = lax.rsqrt(h2_ref[...] + 1.0)
    y1_ref[...] = jnp.dot(x1_ref[...], w1_ref[...],
                          preferred_element_type=jnp.float32) * d1
    y2_ref[...] = jnp.dot(x2_ref[...], w2_ref[...],
                          preferred_element_type=jnp.float32) * d2


def _make_xw_kernel(n_pad, d, out):
    rb = 512
    grid = (n_pad // rb,)
    row = lambda i: (i, 0)
    fixed = lambda i: (0, 0)
    return pl.pallas_call(
        _xw_body,
        grid=grid,
        in_specs=[
            pl.BlockSpec((rb, d), row),
            pl.BlockSpec((d, out), fixed),
            pl.BlockSpec((rb, 128), row),
            pl.BlockSpec((rb, d), row),
            pl.BlockSpec((d, out), fixed),
            pl.BlockSpec((rb, 128), row),
        ],
        out_specs=[
            pl.BlockSpec((rb, out), row),
            pl.BlockSpec((rb, out), row),
        ],
        out_shape=[jax.ShapeDtypeStruct((n_pad, out), jnp.float32)] * 2,
    )


def _make_head_kernel(n_pad, out, n_graphs):
    rb = 512
    nblk = n_pad // rb

    def body(agg1_ref, h1_ref, bv1_ref, agg2_ref, h2_ref, bv2_ref,
             bc1_ref, bc2_ref, wp1_ref, bp1_ref, wp2_ref, bp2_ref,
             w1_ref, b1_ref, w2_ref, b2_ref, wo_ref, bo_ref, o_ref,
             pool1, pool2):
        i = pl.program_id(0)

        @pl.when(i == 0)
        def _():
            pool1[...] = jnp.full((n_graphs + 1, out), -jnp.inf, jnp.float32)
            pool2[...] = jnp.full((n_graphs + 1, out), -jnp.inf, jnp.float32)

        def accum(agg_ref, h_ref, bv_ref, bc_ref, pool):
            dinv = lax.rsqrt(h_ref[...] + 1.0)
            t = _leaky(dinv * agg_ref[...] + bc_ref[...])
            bv = bv_ref[...]
            g_lo = jnp.min(bv)
            g_hi = jnp.max(bv)

            def upd(g, carry):
                m = bv == g
                cand = jnp.max(jnp.where(m, t, -jnp.inf), axis=0,
                               keepdims=True)
                pool[pl.ds(g, 1), :] = jnp.maximum(pool[pl.ds(g, 1), :], cand)
                return carry

            lax.fori_loop(g_lo, g_hi + 1, upd, 0)

        accum(agg1_ref, h1_ref, bv1_ref, bc1_ref, pool1)
        accum(agg2_ref, h2_ref, bv2_ref, bc2_ref, pool2)

        @pl.when(i == nblk - 1)
        def _():
            p1 = pool1[0:n_graphs, :]
            p1 = jnp.where(jnp.isfinite(p1), p1, 0.0)
            p2 = pool2[0:n_graphs, :]
            p2 = jnp.where(jnp.isfinite(p2), p2, 0.0)
            p1 = _leaky(jnp.dot(p1, wp1_ref[...],
                                preferred_element_type=jnp.float32)
                        + bp1_ref[...])
            p2 = _leaky(jnp.dot(p2, wp2_ref[...],
                                preferred_element_type=jnp.float32)
                        + bp2_ref[...])
            cc = jnp.concatenate([p1, p2], axis=1)
            cc = _leaky(jnp.dot(cc, w1_ref[...],
                                preferred_element_type=jnp.float32)
                        + b1_ref[...])
            cc = _leaky(jnp.dot(cc, w2_ref[...],
                                preferred_element_type=jnp.float32)
                        + b2_ref[...])
            z = jnp.dot(cc, wo_ref[...],
                        preferred_element_type=jnp.float32) + bo_ref[...]
            o_ref[...] = 1.0 / (1.0 + jnp.exp(-z))

    row = lambda i: (i, 0)
    fixed = lambda i: (0, 0)
    per_graph = [
        pl.BlockSpec((rb, out), row),
        pl.BlockSpec((rb, 128), row),
        pl.BlockSpec((rb, 1), row),
    ]
    return pl.pallas_call(
        body,
        grid=(nblk,),
        in_specs=per_graph + per_graph + [
            pl.BlockSpec((1, out), fixed),
            pl.BlockSpec((1, out), fixed),
            pl.BlockSpec((out, out), fixed),
            pl.BlockSpec((1, out), fixed),
            pl.BlockSpec((out, out), fixed),
            pl.BlockSpec((1, out), fixed),
            pl.BlockSpec((2 * out, 256), fixed),
            pl.BlockSpec((1, 256), fixed),
            pl.BlockSpec((256, 64), fixed),
            pl.BlockSpec((1, 64), fixed),
            pl.BlockSpec((64, 1), fixed),
            pl.BlockSpec((1, 1), fixed),
        ],
        out_specs=pl.BlockSpec((n_graphs, 1), fixed),
        out_shape=jax.ShapeDtypeStruct((n_graphs, 1), jnp.float32),
        scratch_shapes=[
            pltpu.VMEM((n_graphs + 1, out), jnp.float32),
            pltpu.VMEM((n_graphs + 1, out), jnp.float32),
        ],
    )


def kernel(pro1_x, pro1_edge_index, pro1_batch, pro2_x, pro2_edge_index,
           pro2_batch, W_conv1, b_conv1, W_fc_p1, b_fc_p1, W_conv2, b_conv2,
           W_fc_p2, b_fc_p2, W_fc1, b_fc1, W_fc2, b_fc2, W_out, b_out):
    n, d = pro1_x.shape
    out = W_conv1.shape[1]
    e = pro1_edge_index.shape[1]
    n_graphs = 64

    n_pad = -(-n // (NS * 128)) * (NS * 128)
    scrap = n
    e_pad = -(-e // (NS * CK * GS)) * (NS * CK * GS)
    e_rows = e_pad // CK

    i32 = jnp.int32
    f32 = jnp.float32

    def prep_graph(x, ei, batch):
        ei = ei.astype(i32)
        pad_e = e_pad - e
        src = jnp.concatenate([ei[0], jnp.zeros((pad_e,), i32)])
        dst = jnp.concatenate([ei[1], jnp.full((pad_e,), scrap, i32)])
        src = src.reshape(e_rows, CK)
        dst = dst.reshape(e_rows, CK)
        xp = jnp.concatenate([x, jnp.zeros((n_pad - n, d), f32)], axis=0)
        bp = jnp.concatenate(
            [batch.astype(i32), jnp.full((n_pad - n,), n_graphs, i32)]
        ).reshape(n_pad, 1)
        return src, dst, xp, bp

    src1, dst1, x1p, b1p = prep_graph(pro1_x, pro1_edge_index, pro1_batch)
    src2, dst2, x2p, b2p = prep_graph(pro2_x, pro2_edge_index, pro2_batch)

    h1, h2 = _make_deg_kernel(n_pad, e_rows)(dst1, dst2)
    y1, y2 = _make_xw_kernel(n_pad, d, out)(x1p, W_conv1, h1, x2p, W_conv2,
                                            h2)
    agg1, agg2 = _make_agg_kernel(n_pad, e_rows, out)(src1, dst1, y1, src2,
                                                      dst2, y2)

    res = _make_head_kernel(n_pad, out, n_graphs)(
        agg1, h1, b1p, agg2, h2, b2p,
        b_conv1.reshape(1, out), b_conv2.reshape(1, out),
        W_fc_p1, b_fc_p1.reshape(1, out), W_fc_p2, b_fc_p2.reshape(1, out),
        W_fc1, b_fc1.reshape(1, 256), W_fc2, b_fc2.reshape(1, 64),
        W_out, b_out.reshape(1, 1))
    return res


# speedup vs baseline: 1.6970x; 1.5380x over previous
import functools

import jax
import jax.numpy as jnp
from jax import lax
from jax.experimental import pallas as pl
from jax.experimental.pallas import tpu as pltpu
from jax.experimental.pallas import tpu_sc as plsc

NS = 16
CK = 112
NI = 6


def _leaky(v):
    return jnp.where(v >= 0, v, 0.01 * v)


def _make_deg_kernel(n_pad, e_rows):
    rows_t = n_pad // NS
    ert = e_rows // NS
    mesh = plsc.VectorSubcoreMesh(core_axis_name="c", subcore_axis_name="s")
    f32 = jnp.float32

    def body(dst1, dst2, d1, d2, idx0_v, idx1_v, tab_v, red_v, degb_v, red_sh,
             sem0, sem1):
        s = lax.axis_index("s")
        c = lax.axis_index("c")

        def run(dst, d_out):
            def z(i, carry):
                tab_v[pl.ds(i * 16, 16)] = jnp.zeros((16,), f32)
                return carry

            lax.fori_loop(0, n_pad // 16, z, 0)
            base = s * ert
            pltpu.sync_copy(dst.at[base], idx0_v)
            pltpu.async_copy(dst.at[base + 1], idx1_v, sem1)

            def scat(idx_v):
                def sc16(i, carry2):
                    iv = idx_v[pl.ds(i * 16, 16)]
                    plsc.addupdate_scatter(tab_v, [iv],
                                           jnp.ones((16,), f32))
                    return carry2

                lax.fori_loop(0, CK // 16, sc16, 0)

            def pair(t, carry):
                c0 = 2 * t
                scat(idx0_v)

                @pl.when(c0 + 2 < ert)
                def _():
                    pltpu.async_copy(dst.at[base + c0 + 2], idx0_v, sem0)

                pltpu.make_async_copy(dst.at[base], idx1_v, sem1).wait()
                scat(idx1_v)

                @pl.when(c0 + 3 < ert)
                def _():
                    pltpu.async_copy(dst.at[base + c0 + 3], idx1_v, sem1)

                @pl.when(c0 + 2 < ert)
                def _():
                    pltpu.make_async_copy(dst.at[base], idx0_v, sem0).wait()

                return carry

            lax.fori_loop(0, ert // 2, pair, 0)
            pltpu.sync_copy(tab_v, red_sh.at[s])
            plsc.subcore_barrier()
            pltpu.sync_copy(red_sh.at[:, pl.ds(s * rows_t, rows_t)], red_v)

            def red(cix, carry):
                acc = jnp.zeros((16,), f32)
                for t in range(NS):
                    acc = acc + red_v[t, pl.ds(cix * 16, 16)]
                tab_v[pl.ds(cix * 16, 16)] = acc
                return carry

            lax.fori_loop(0, rows_t // 16, red, 0)

            def bc(nix, carry):
                vec = plsc.load_gather(tab_v, [jnp.full((16,), nix,
                                                        jnp.int32)])
                for cix in range(8):
                    degb_v[nix, pl.ds(cix * 16, 16)] = vec
                return carry

            lax.fori_loop(0, rows_t, bc, 0)
            pltpu.sync_copy(degb_v, d_out.at[pl.ds(s * rows_t, rows_t)])

        @pl.when(c == 0)
        def _():
            run(dst1, d1)

        @pl.when(c == 1)
        def _():
            run(dst2, d2)

    return pl.kernel(
        body,
        out_type=[jax.ShapeDtypeStruct((n_pad, 128), jnp.float32)] * 2,
        mesh=mesh,
        scratch_types=[
            pltpu.VMEM((CK,), jnp.int32),
            pltpu.VMEM((CK,), jnp.int32),
            pltpu.VMEM((n_pad,), jnp.float32),
            pltpu.VMEM((NS, n_pad // NS), jnp.float32),
            pltpu.VMEM((n_pad // NS, 128), jnp.float32),
            pltpu.VMEM_SHARED((NS, n_pad), jnp.float32),
            pltpu.SemaphoreType.DMA,
            pltpu.SemaphoreType.DMA,
        ],
        compiler_params=pltpu.CompilerParams(needs_layout_passes=False),
    )


def _make_agg_kernel(n_pad, e_rows, d):
    rows_t = n_pad // NS
    ert = e_rows // NS
    mesh = plsc.VectorSubcoreMesh(core_axis_name="c", subcore_axis_name="s")

    NB = 3

    def body(src1, dst1, y1, src2, dst2, y2, agg1, agg2, sidx, didx,
             rows, agg_sh, sem_g, sem_s, sem_i):
        s = lax.axis_index("s")
        c = lax.axis_index("c")

        def run(src, dst, y, agg_out):
            pltpu.sync_copy(y.at[pl.ds(s * rows_t, rows_t)],
                            agg_sh.at[pl.ds(s * rows_t, rows_t)])
            plsc.subcore_barrier()

            base = s * ert
            for k in range(NB):
                pltpu.sync_copy(src.at[base + k], sidx[k])
                pltpu.sync_copy(dst.at[base + k], didx[k])
                pltpu.async_copy(y.at[sidx[k]], rows[k], sem_g[k])

            def octet(t, carry):
                for k in range(NI):
                    ck = NI * t + k
                    srot = k % NB
                    prot = (k + NB) % NI
                    pltpu.make_async_copy(y.at[sidx[k]], rows[srot],
                                          sem_g[srot]).wait()
                    sc = pltpu.async_copy(rows[srot], agg_sh.at[didx[k]],
                                          sem_s, add=True)

                    @pl.when(ck + NB < ert)
                    def _():
                        pltpu.async_copy(src.at[base + ck + NB], sidx[prot],
                                         sem_i)
                        pltpu.async_copy(dst.at[base + ck + NB], didx[prot],
                                         sem_i)

                    sc.wait()

                    @pl.when(ck + NB < ert)
                    def _():
                        pltpu.make_async_copy(src.at[base], sidx[prot],
                                              sem_i).wait()
                        pltpu.make_async_copy(dst.at[base], didx[prot],
                                              sem_i).wait()
                        pltpu.async_copy(y.at[sidx[prot]], rows[srot],
                                         sem_g[srot])

                return carry

            lax.fori_loop(0, ert // NI, octet, 0)
            plsc.subcore_barrier()
            pltpu.sync_copy(agg_sh.at[pl.ds(s * rows_t, rows_t)],
                            agg_out.at[pl.ds(s * rows_t, rows_t)])

        @pl.when(c == 0)
        def _():
            run(src1, dst1, y1, agg1)

        @pl.when(c == 1)
        def _():
            run(src2, dst2, y2, agg2)

    return pl.kernel(
        body,
        out_type=[jax.ShapeDtypeStruct((n_pad, d), jnp.float32)] * 2,
        mesh=mesh,
        scratch_types=[
            [pltpu.VMEM((CK,), jnp.int32)] * NI,
            [pltpu.VMEM((CK,), jnp.int32)] * NI,
            [pltpu.VMEM((CK, d), jnp.float32)] * NB,
            pltpu.VMEM_SHARED((n_pad, d), jnp.float32),
            [pltpu.SemaphoreType.DMA] * NB,
            pltpu.SemaphoreType.DMA,
            pltpu.SemaphoreType.DMA,
        ],
    )


def _xw_body(x1_ref, w1_ref, h1_ref, x2_ref, w2_ref, h2_ref, y1_ref, y2_ref):
    d1 = lax.rsqrt(h1_ref[...] + 1.0)
    d2 = lax.rsqrt(h2_ref[...] + 1.0)
    y1_ref[...] = jnp.dot(x1_ref[...], w1_ref[...],
                          preferred_element_type=jnp.float32) * d1
    y2_ref[...] = jnp.dot(x2_ref[...], w2_ref[...],
                          preferred_element_type=jnp.float32) * d2


def _make_xw_kernel(n_pad, d, out):
    rb = 512
    grid = (n_pad // rb,)
    row = lambda i: (i, 0)
    fixed = lambda i: (0, 0)
    return pl.pallas_call(
        _xw_body,
        grid=grid,
        in_specs=[
            pl.BlockSpec((rb, d), row),
            pl.BlockSpec((d, out), fixed),
            pl.BlockSpec((rb, 128), row),
            pl.BlockSpec((rb, d), row),
            pl.BlockSpec((d, out), fixed),
            pl.BlockSpec((rb, 128), row),
        ],
        out_specs=[
            pl.BlockSpec((rb, out), row),
            pl.BlockSpec((rb, out), row),
        ],
        out_shape=[jax.ShapeDtypeStruct((n_pad, out), jnp.float32)] * 2,
    )


def _make_head_kernel(n_pad, out, n_graphs):
    rb = 512
    nblk = n_pad // rb

    def body(agg1_ref, h1_ref, bv1_ref, agg2_ref, h2_ref, bv2_ref,
             bc1_ref, bc2_ref, wp1_ref, bp1_ref, wp2_ref, bp2_ref,
             w1_ref, b1_ref, w2_ref, b2_ref, wo_ref, bo_ref, o_ref,
             pool1, pool2):
        i = pl.program_id(0)

        @pl.when(i == 0)
        def _():
            pool1[...] = jnp.full((n_graphs + 1, out), -jnp.inf, jnp.float32)
            pool2[...] = jnp.full((n_graphs + 1, out), -jnp.inf, jnp.float32)

        def accum(agg_ref, h_ref, bv_ref, bc_ref, pool):
            dinv = lax.rsqrt(h_ref[...] + 1.0)
            t = _leaky(dinv * agg_ref[...] + bc_ref[...])
            bv = bv_ref[...]
            g_lo = jnp.min(bv)
            g_hi = jnp.max(bv)

            def upd(g, carry):
                m = bv == g
                cand = jnp.max(jnp.where(m, t, -jnp.inf), axis=0,
                               keepdims=True)
                pool[pl.ds(g, 1), :] = jnp.maximum(pool[pl.ds(g, 1), :], cand)
                return carry

            lax.fori_loop(g_lo, g_hi + 1, upd, 0)

        accum(agg1_ref, h1_ref, bv1_ref, bc1_ref, pool1)
        accum(agg2_ref, h2_ref, bv2_ref, bc2_ref, pool2)

        @pl.when(i == nblk - 1)
        def _():
            p1 = pool1[0:n_graphs, :]
            p1 = jnp.where(jnp.isfinite(p1), p1, 0.0)
            p2 = pool2[0:n_graphs, :]
            p2 = jnp.where(jnp.isfinite(p2), p2, 0.0)
            p1 = _leaky(jnp.dot(p1, wp1_ref[...],
                                preferred_element_type=jnp.float32)
                        + bp1_ref[...])
            p2 = _leaky(jnp.dot(p2, wp2_ref[...],
                                preferred_element_type=jnp.float32)
                        + bp2_ref[...])
            cc = jnp.concatenate([p1, p2], axis=1)
            cc = _leaky(jnp.dot(cc, w1_ref[...],
                                preferred_element_type=jnp.float32)
                        + b1_ref[...])
            cc = _leaky(jnp.dot(cc, w2_ref[...],
                                preferred_element_type=jnp.float32)
                        + b2_ref[...])
            z = jnp.dot(cc, wo_ref[...],
                        preferred_element_type=jnp.float32) + bo_ref[...]
            o_ref[...] = 1.0 / (1.0 + jnp.exp(-z))

    row = lambda i: (i, 0)
    fixed = lambda i: (0, 0)
    per_graph = [
        pl.BlockSpec((rb, out), row),
        pl.BlockSpec((rb, 128), row),
        pl.BlockSpec((rb, 1), row),
    ]
    return pl.pallas_call(
        body,
        grid=(nblk,),
        in_specs=per_graph + per_graph + [
            pl.BlockSpec((1, out), fixed),
            pl.BlockSpec((1, out), fixed),
            pl.BlockSpec((out, out), fixed),
            pl.BlockSpec((1, out), fixed),
            pl.BlockSpec((out, out), fixed),
            pl.BlockSpec((1, out), fixed),
            pl.BlockSpec((2 * out, 256), fixed),
            pl.BlockSpec((1, 256), fixed),
            pl.BlockSpec((256, 64), fixed),
            pl.BlockSpec((1, 64), fixed),
            pl.BlockSpec((64, 1), fixed),
            pl.BlockSpec((1, 1), fixed),
        ],
        out_specs=pl.BlockSpec((n_graphs, 1), fixed),
        out_shape=jax.ShapeDtypeStruct((n_graphs, 1), jnp.float32),
        scratch_shapes=[
            pltpu.VMEM((n_graphs + 1, out), jnp.float32),
            pltpu.VMEM((n_graphs + 1, out), jnp.float32),
        ],
    )


def kernel(pro1_x, pro1_edge_index, pro1_batch, pro2_x, pro2_edge_index,
           pro2_batch, W_conv1, b_conv1, W_fc_p1, b_fc_p1, W_conv2, b_conv2,
           W_fc_p2, b_fc_p2, W_fc1, b_fc1, W_fc2, b_fc2, W_out, b_out):
    n, d = pro1_x.shape
    out = W_conv1.shape[1]
    e = pro1_edge_index.shape[1]
    n_graphs = 64

    n_pad = -(-n // (NS * 128)) * (NS * 128)
    scrap = n
    ert = -(-e // (NS * CK))
    ert = -(-ert // NI) * NI
    e_pad = ert * NS * CK
    e_rows = e_pad // CK

    i32 = jnp.int32
    f32 = jnp.float32

    def prep_graph(x, ei, batch):
        ei = ei.astype(i32)
        pad_e = e_pad - e
        src = jnp.concatenate([ei[0], jnp.zeros((pad_e,), i32)])
        dst = jnp.concatenate([ei[1], jnp.full((pad_e,), scrap, i32)])
        src = src.reshape(e_rows, CK)
        dst = dst.reshape(e_rows, CK)
        xp = jnp.concatenate([x, jnp.zeros((n_pad - n, d), f32)], axis=0)
        bp = jnp.concatenate(
            [batch.astype(i32), jnp.full((n_pad - n,), n_graphs, i32)]
        ).reshape(n_pad, 1)
        return src, dst, xp, bp

    src1, dst1, x1p, b1p = prep_graph(pro1_x, pro1_edge_index, pro1_batch)
    src2, dst2, x2p, b2p = prep_graph(pro2_x, pro2_edge_index, pro2_batch)

    h1, h2 = _make_deg_kernel(n_pad, e_rows)(dst1, dst2)
    y1, y2 = _make_xw_kernel(n_pad, d, out)(x1p, W_conv1, h1, x2p, W_conv2,
                                            h2)
    agg1, agg2 = _make_agg_kernel(n_pad, e_rows, out)(src1, dst1, y1, src2,
                                                      dst2, y2)

    res = _make_head_kernel(n_pad, out, n_graphs)(
        agg1, h1, b1p, agg2, h2, b2p,
        b_conv1.reshape(1, out), b_conv2.reshape(1, out),
        W_fc_p1, b_fc_p1.reshape(1, out), W_fc_p2, b_fc_p2.reshape(1, out),
        W_fc1, b_fc1.reshape(1, 256), W_fc2, b_fc2.reshape(1, 64),
        W_out, b_out.reshape(1, 1))
    return res
